# Initial kernel scaffold; baseline (speedup 1.0000x reference)
import functools

import jax
import jax.numpy as jnp
from jax import lax
from jax.experimental import pallas as pl
from jax.experimental.pallas import tpu as pltpu
from jax.experimental.pallas import tpu_sc as plsc

NC = 2
NS = 16
NW = NC * NS
LANES = 16
CHUNK = 128
BR = 256


def _sc_mesh():
    return plsc.VectorSubcoreMesh(core_axis_name="c", subcore_axis_name="s")


def _make_deg_kernel(n_pad, cpw):
    rps = n_pad // NS

    @functools.partial(
        pl.kernel,
        out_type=jax.ShapeDtypeStruct((NC, n_pad), jnp.float32),
        mesh=_sc_mesh(),
        scratch_types=[
            pltpu.VMEM((cpw, CHUNK), jnp.int32),
            pltpu.VMEM((CHUNK,), jnp.float32),
            pltpu.VMEM_SHARED((n_pad,), jnp.float32),
            pltpu.SemaphoreType.DMA,
        ],
    )
    def deg_kernel(dst_hbm, out_hbm, didx_v, ones_v, deg_sh, sem):
        c = lax.axis_index("c")
        s = lax.axis_index("s")
        w = c * NS + s
        base = pl.multiple_of(s * rps, 8)

        @pl.loop(0, CHUNK, step=LANES)
        def _(i):
            ones_v[pl.ds(i, LANES)] = jnp.zeros((LANES,), jnp.float32)

        for k in range(rps // CHUNK):
            pltpu.sync_copy(ones_v, deg_sh.at[pl.ds(base + k * CHUNK, CHUNK)])

        @pl.loop(0, CHUNK, step=LANES)
        def _(i):
            ones_v[pl.ds(i, LANES)] = jnp.ones((LANES,), jnp.float32)

        pltpu.sync_copy(dst_hbm.at[w], didx_v)
        plsc.subcore_barrier()

        @pl.loop(0, cpw)
        def _(j):
            pltpu.sync_copy(ones_v, deg_sh.at[didx_v.at[j]], add=True)

        plsc.subcore_barrier()
        pltpu.sync_copy(deg_sh.at[pl.ds(base, rps)],
                        out_hbm.at[c, pl.ds(base, rps)])

    return deg_kernel


def _make_agg_kernel(n_pad, cpw):
    rps = n_pad // NS

    @functools.partial(
        pl.kernel,
        out_type=jax.ShapeDtypeStruct((NC, n_pad, LANES), jnp.float32),
        mesh=_sc_mesh(),
        scratch_types=[
            pltpu.VMEM((cpw, CHUNK), jnp.int32),
            pltpu.VMEM((cpw, CHUNK), jnp.int32),
            pltpu.VMEM((CHUNK, LANES), jnp.float32),
            pltpu.VMEM_SHARED((n_pad, LANES), jnp.float32),
            pltpu.SemaphoreType.DMA,
        ],
    )
    def agg_kernel(g_hbm, src_hbm, dst_hbm, out_hbm,
                   sidx_v, didx_v, rows_v, agg_sh, sem):
        c = lax.axis_index("c")
        s = lax.axis_index("s")
        w = c * NS + s
        base = pl.multiple_of(s * rps, 8)

        @pl.loop(0, CHUNK)
        def _(i):
            rows_v[i, :] = jnp.zeros((LANES,), jnp.float32)

        for k in range(rps // CHUNK):
            pltpu.sync_copy(rows_v, agg_sh.at[pl.ds(base + k * CHUNK, CHUNK)])

        pltpu.sync_copy(src_hbm.at[w], sidx_v)
        pltpu.sync_copy(dst_hbm.at[w], didx_v)
        plsc.subcore_barrier()

        @pl.loop(0, cpw)
        def _(j):
            pltpu.async_copy(g_hbm.at[sidx_v.at[j]], rows_v, sem).wait()
            pltpu.sync_copy(rows_v, agg_sh.at[didx_v.at[j]], add=True)

        plsc.subcore_barrier()
        pltpu.sync_copy(agg_sh.at[pl.ds(base, rps)],
                        out_hbm.at[c, pl.ds(base, rps)])

    return agg_kernel


def _g1_body(x_ref, w1_ref, degp_ref, g1_ref, dinv_ref):
    deg = degp_ref[0, :] + degp_ref[1, :] + 1.0
    dinv = lax.rsqrt(deg)
    h = jnp.dot(x_ref[...], w1_ref[...], preferred_element_type=jnp.float32)
    g1_ref[...] = h * dinv[:, None]
    dinv_ref[...] = dinv


def _g2_body(p_ref, g1_ref, dinv_ref, b1_ref, g2_ref):
    dinv = dinv_ref[...][:, None]
    agg = p_ref[0] + p_ref[1] + g1_ref[...]
    out1 = agg * dinv + b1_ref[...][None, :]
    g2_ref[...] = jnp.maximum(out1, 0.0) * dinv


def _out_body(p_ref, g2_ref, dinv_ref, w2_ref, b2_ref, o_ref):
    dinv = dinv_ref[...][:, None]
    agg = (p_ref[0] + p_ref[1] + g2_ref[...]) * dinv
    v = jnp.dot(agg, w2_ref[...], preferred_element_type=jnp.float32)
    v = v + b2_ref[...][None, :]
    m = jnp.max(v, axis=1, keepdims=True)
    lse = m + jnp.log(jnp.sum(jnp.exp(v - m), axis=1, keepdims=True))
    o_ref[...] = v - lse


def _g1_call(x_pad, w1, degp):
    n_pad, d = x_pad.shape
    h = w1.shape[1]
    return pl.pallas_call(
        _g1_body,
        grid=(n_pad // BR,),
        in_specs=[
            pl.BlockSpec((BR, d), lambda i: (i, 0)),
            pl.BlockSpec((d, h), lambda i: (0, 0)),
            pl.BlockSpec((NC, BR), lambda i: (0, i)),
        ],
        out_specs=[
            pl.BlockSpec((BR, h), lambda i: (i, 0)),
            pl.BlockSpec((BR,), lambda i: (i,)),
        ],
        out_shape=[
            jax.ShapeDtypeStruct((n_pad, h), jnp.float32),
            jax.ShapeDtypeStruct((n_pad,), jnp.float32),
        ],
    )(x_pad, w1, degp)


def _g2_call(p, g1, dinv, b1):
    n_pad, h = g1.shape
    return pl.pallas_call(
        _g2_body,
        grid=(n_pad // BR,),
        in_specs=[
            pl.BlockSpec((NC, BR, h), lambda i: (0, i, 0)),
            pl.BlockSpec((BR, h), lambda i: (i, 0)),
            pl.BlockSpec((BR,), lambda i: (i,)),
            pl.BlockSpec((h,), lambda i: (0,)),
        ],
        out_specs=pl.BlockSpec((BR, h), lambda i: (i, 0)),
        out_shape=jax.ShapeDtypeStruct((n_pad, h), jnp.float32),
    )(p, g1, dinv, b1)


def _out_call(p, g2, dinv, w2, b2):
    n_pad, h = g2.shape
    c = w2.shape[1]
    return pl.pallas_call(
        _out_body,
        grid=(n_pad // BR,),
        in_specs=[
            pl.BlockSpec((NC, BR, h), lambda i: (0, i, 0)),
            pl.BlockSpec((BR, h), lambda i: (i, 0)),
            pl.BlockSpec((BR,), lambda i: (i,)),
            pl.BlockSpec((h, c), lambda i: (0, 0)),
            pl.BlockSpec((c,), lambda i: (0,)),
        ],
        out_specs=pl.BlockSpec((BR, c), lambda i: (i, 0)),
        out_shape=jax.ShapeDtypeStruct((n_pad, c), jnp.float32),
    )(p, g2, dinv, w2, b2)


def kernel(x, edge_index, W1, b1, W2, b2):
    n, d = x.shape
    e = edge_index.shape[1]
    n_pad = ((n + 1 + NS * CHUNK - 1) // (NS * CHUNK)) * (NS * CHUNK)
    cpw = (e + NW * CHUNK - 1) // (NW * CHUNK)
    e_pad = NW * cpw * CHUNK

    pad = jnp.full((e_pad - e,), n, dtype=jnp.int32)
    src_r = jnp.concatenate([edge_index[0], pad]).reshape(NW, cpw, CHUNK)
    dst_r = jnp.concatenate([edge_index[1], pad]).reshape(NW, cpw, CHUNK)
    x_pad = jnp.pad(x, ((0, n_pad - n), (0, 0)))

    agg = _make_agg_kernel(n_pad, cpw)
    degp = _make_deg_kernel(n_pad, cpw)(dst_r)

    g1, dinv = _g1_call(x_pad, W1, degp)
    p1 = agg(g1, src_r, dst_r)
    g2 = _g2_call(p1, g1, dinv, b1)
    p2 = agg(g2, src_r, dst_r)
    out = _out_call(p2, g2, dinv, W2, b2)
    return out[:n]


# speedup vs baseline: 29.0311x; 29.0311x over previous
import functools

import jax
import jax.numpy as jnp
from jax import lax
from jax.experimental import pallas as pl
from jax.experimental.pallas import tpu as pltpu
from jax.experimental.pallas import tpu_sc as plsc

NC = 2
NS = 16
NW = NC * NS
LANES = 16
CHUNK = 128
BR = 256


def _sc_mesh():
    return plsc.VectorSubcoreMesh(core_axis_name="c", subcore_axis_name="s")


def _make_deg_kernel(n_pad, cpw):
    rps = n_pad // NS

    @functools.partial(
        pl.kernel,
        out_type=jax.ShapeDtypeStruct((NC, n_pad), jnp.float32),
        mesh=_sc_mesh(),
        scratch_types=[
            pltpu.VMEM((cpw, CHUNK), jnp.int32),
            pltpu.VMEM((CHUNK,), jnp.float32),
            pltpu.VMEM_SHARED((n_pad,), jnp.float32),
            pltpu.SemaphoreType.DMA,
        ],
    )
    def deg_kernel(dst_hbm, out_hbm, didx_v, ones_v, deg_sh, sem):
        c = lax.axis_index("c")
        s = lax.axis_index("s")
        w = c * NS + s
        base = pl.multiple_of(s * rps, 8)

        @pl.loop(0, CHUNK, step=LANES)
        def _(i):
            ones_v[pl.ds(i, LANES)] = jnp.zeros((LANES,), jnp.float32)

        for k in range(rps // CHUNK):
            pltpu.sync_copy(ones_v, deg_sh.at[pl.ds(base + k * CHUNK, CHUNK)])

        @pl.loop(0, CHUNK, step=LANES)
        def _(i):
            ones_v[pl.ds(i, LANES)] = jnp.ones((LANES,), jnp.float32)

        pltpu.sync_copy(dst_hbm.at[w], didx_v)
        plsc.subcore_barrier()

        @pl.loop(0, cpw)
        def _(j):
            pltpu.sync_copy(ones_v, deg_sh.at[didx_v.at[j]], add=True)

        plsc.subcore_barrier()
        pltpu.sync_copy(deg_sh.at[pl.ds(base, rps)],
                        out_hbm.at[c, pl.ds(base, rps)])

    return deg_kernel


def _make_agg_kernel(n_pad, cpw):
    rps = n_pad // NS

    @functools.partial(
        pl.kernel,
        out_type=jax.ShapeDtypeStruct((NC, n_pad, LANES), jnp.float32),
        mesh=_sc_mesh(),
        scratch_types=[
            pltpu.VMEM((cpw, CHUNK), jnp.int32),
            pltpu.VMEM((cpw, CHUNK), jnp.int32),
            pltpu.VMEM((CHUNK, LANES), jnp.float32),
            pltpu.VMEM_SHARED((n_pad, LANES), jnp.float32),
            pltpu.SemaphoreType.DMA,
        ],
        compiler_params=pltpu.CompilerParams(use_tc_tiling_on_sc=False),
    )
    def agg_kernel(g_hbm, src_hbm, dst_hbm, out_hbm,
                   sidx_v, didx_v, rows_v, agg_sh, sem):
        c = lax.axis_index("c")
        s = lax.axis_index("s")
        w = c * NS + s
        base = pl.multiple_of(s * rps, 8)

        @pl.loop(0, CHUNK)
        def _(i):
            rows_v[i, :] = jnp.zeros((LANES,), jnp.float32)

        for k in range(rps // CHUNK):
            pltpu.sync_copy(rows_v, agg_sh.at[pl.ds(base + k * CHUNK, CHUNK)])

        pltpu.sync_copy(src_hbm.at[w], sidx_v)
        pltpu.sync_copy(dst_hbm.at[w], didx_v)
        plsc.subcore_barrier()

        @pl.loop(0, cpw)
        def _(j):
            pltpu.async_copy(g_hbm.at[sidx_v.at[j]], rows_v, sem).wait()
            pltpu.sync_copy(rows_v, agg_sh.at[didx_v.at[j]], add=True)

        plsc.subcore_barrier()
        pltpu.sync_copy(agg_sh.at[pl.ds(base, rps)],
                        out_hbm.at[c, pl.ds(base, rps)])

    return agg_kernel


def _g1_body(x_ref, w1_ref, degp_ref, g1_ref, dinv_ref):
    deg = degp_ref[0, :] + degp_ref[1, :] + 1.0
    dinv = lax.rsqrt(deg)
    h = jnp.dot(x_ref[...], w1_ref[...], preferred_element_type=jnp.float32)
    g1_ref[...] = h * dinv[:, None]
    dinv_ref[...] = dinv


def _g2_body(p_ref, g1_ref, dinv_ref, b1_ref, g2_ref):
    dinv = dinv_ref[...][:, None]
    agg = p_ref[0] + p_ref[1] + g1_ref[...]
    out1 = agg * dinv + b1_ref[...][None, :]
    g2_ref[...] = jnp.maximum(out1, 0.0) * dinv


def _out_body(p_ref, g2_ref, dinv_ref, w2_ref, b2_ref, o_ref):
    dinv = dinv_ref[...][:, None]
    agg = (p_ref[0] + p_ref[1] + g2_ref[...]) * dinv
    v = jnp.dot(agg, w2_ref[...], preferred_element_type=jnp.float32)
    v = v + b2_ref[...][None, :]
    m = jnp.max(v, axis=1, keepdims=True)
    lse = m + jnp.log(jnp.sum(jnp.exp(v - m), axis=1, keepdims=True))
    o_ref[...] = v - lse


def _g1_call(x_pad, w1, degp):
    n_pad, d = x_pad.shape
    h = w1.shape[1]
    return pl.pallas_call(
        _g1_body,
        grid=(n_pad // BR,),
        in_specs=[
            pl.BlockSpec((BR, d), lambda i: (i, 0)),
            pl.BlockSpec((d, h), lambda i: (0, 0)),
            pl.BlockSpec((NC, BR), lambda i: (0, i)),
        ],
        out_specs=[
            pl.BlockSpec((BR, h), lambda i: (i, 0)),
            pl.BlockSpec((BR,), lambda i: (i,)),
        ],
        out_shape=[
            jax.ShapeDtypeStruct((n_pad, h), jnp.float32),
            jax.ShapeDtypeStruct((n_pad,), jnp.float32),
        ],
    )(x_pad, w1, degp)


def _g2_call(p, g1, dinv, b1):
    n_pad, h = g1.shape
    return pl.pallas_call(
        _g2_body,
        grid=(n_pad // BR,),
        in_specs=[
            pl.BlockSpec((NC, BR, h), lambda i: (0, i, 0)),
            pl.BlockSpec((BR, h), lambda i: (i, 0)),
            pl.BlockSpec((BR,), lambda i: (i,)),
            pl.BlockSpec((h,), lambda i: (0,)),
        ],
        out_specs=pl.BlockSpec((BR, h), lambda i: (i, 0)),
        out_shape=jax.ShapeDtypeStruct((n_pad, h), jnp.float32),
    )(p, g1, dinv, b1)


def _out_call(p, g2, dinv, w2, b2):
    n_pad, h = g2.shape
    c = w2.shape[1]
    return pl.pallas_call(
        _out_body,
        grid=(n_pad // BR,),
        in_specs=[
            pl.BlockSpec((NC, BR, h), lambda i: (0, i, 0)),
            pl.BlockSpec((BR, h), lambda i: (i, 0)),
            pl.BlockSpec((BR,), lambda i: (i,)),
            pl.BlockSpec((h, c), lambda i: (0, 0)),
            pl.BlockSpec((c,), lambda i: (0,)),
        ],
        out_specs=pl.BlockSpec((BR, c), lambda i: (i, 0)),
        out_shape=jax.ShapeDtypeStruct((n_pad, c), jnp.float32),
    )(p, g2, dinv, w2, b2)


def kernel(x, edge_index, W1, b1, W2, b2):
    n, d = x.shape
    e = edge_index.shape[1]
    n_pad = ((n + 1 + NS * CHUNK - 1) // (NS * CHUNK)) * (NS * CHUNK)
    cpw = (e + NW * CHUNK - 1) // (NW * CHUNK)
    e_pad = NW * cpw * CHUNK

    pad = jnp.full((e_pad - e,), n, dtype=jnp.int32)
    src_r = jnp.concatenate([edge_index[0], pad]).reshape(NW, cpw, CHUNK)
    dst_r = jnp.concatenate([edge_index[1], pad]).reshape(NW, cpw, CHUNK)
    x_pad = jnp.pad(x, ((0, n_pad - n), (0, 0)))

    agg = _make_agg_kernel(n_pad, cpw)
    degp = _make_deg_kernel(n_pad, cpw)(dst_r)

    g1, dinv = _g1_call(x_pad, W1, degp)
    p1 = agg(g1, src_r, dst_r)
    g2 = _g2_call(p1, g1, dinv, b1)
    p2 = agg(g2, src_r, dst_r)
    out = _out_call(p2, g2, dinv, W2, b2)
    return out[:n]


# speedup vs baseline: 35.3111x; 1.2163x over previous
import functools

import jax
import jax.numpy as jnp
from jax import lax
from jax.experimental import pallas as pl
from jax.experimental.pallas import tpu as pltpu
from jax.experimental.pallas import tpu_sc as plsc

NC = 2
NS = 16
NW = NC * NS
LANES = 16
CHUNK = 128
NBUF = 4
BR = 256


def _sc_mesh():
    return plsc.VectorSubcoreMesh(core_axis_name="c", subcore_axis_name="s")


def _make_deg_kernel(n_pad, cpw):
    rps = n_pad // NS

    @functools.partial(
        pl.kernel,
        out_type=jax.ShapeDtypeStruct((NC, n_pad), jnp.float32),
        mesh=_sc_mesh(),
        scratch_types=[
            pltpu.VMEM((cpw, CHUNK), jnp.int32),
            pltpu.VMEM((CHUNK,), jnp.float32),
            pltpu.VMEM_SHARED((n_pad,), jnp.float32),
            pltpu.SemaphoreType.DMA,
        ],
    )
    def deg_kernel(dst_hbm, out_hbm, didx_v, ones_v, deg_sh, sem):
        c = lax.axis_index("c")
        s = lax.axis_index("s")
        w = c * NS + s
        base = pl.multiple_of(s * rps, 8)

        @pl.loop(0, CHUNK, step=LANES)
        def _(i):
            ones_v[pl.ds(i, LANES)] = jnp.zeros((LANES,), jnp.float32)

        for k in range(rps // CHUNK):
            pltpu.sync_copy(ones_v, deg_sh.at[pl.ds(base + k * CHUNK, CHUNK)])

        @pl.loop(0, CHUNK, step=LANES)
        def _(i):
            ones_v[pl.ds(i, LANES)] = jnp.ones((LANES,), jnp.float32)

        pltpu.sync_copy(dst_hbm.at[w], didx_v)
        plsc.subcore_barrier()

        @pl.loop(0, cpw)
        def _(j):
            pltpu.sync_copy(ones_v, deg_sh.at[didx_v.at[j]], add=True)

        plsc.subcore_barrier()
        pltpu.sync_copy(deg_sh.at[pl.ds(base, rps)],
                        out_hbm.at[c, pl.ds(base, rps)])

    return deg_kernel


def _make_agg_kernel(n_pad, cpw):
    rps = n_pad // NS
    assert cpw % NBUF == 0 and cpw // NBUF >= 2
    ngroups = cpw // NBUF

    @functools.partial(
        pl.kernel,
        out_type=jax.ShapeDtypeStruct((NC, n_pad, LANES), jnp.float32),
        mesh=_sc_mesh(),
        scratch_types=[
            pltpu.VMEM((cpw, CHUNK), jnp.int32),
            pltpu.VMEM((cpw, CHUNK), jnp.int32),
            pltpu.VMEM((NBUF, CHUNK, LANES), jnp.float32),
            pltpu.VMEM_SHARED((n_pad, LANES), jnp.float32),
        ] + [pltpu.SemaphoreType.DMA] * NBUF,
        compiler_params=pltpu.CompilerParams(use_tc_tiling_on_sc=False),
    )
    def agg_kernel(g_hbm, src_hbm, dst_hbm, out_hbm,
                   sidx_v, didx_v, rows_v, agg_sh, *sems):
        c = lax.axis_index("c")
        s = lax.axis_index("s")
        w = c * NS + s
        base = pl.multiple_of(s * rps, 8)

        @pl.loop(0, CHUNK)
        def _(i):
            rows_v[0, i, :] = jnp.zeros((LANES,), jnp.float32)

        for k in range(rps // CHUNK):
            pltpu.sync_copy(rows_v.at[0], agg_sh.at[pl.ds(base + k * CHUNK, CHUNK)])

        pltpu.sync_copy(src_hbm.at[w], sidx_v)
        pltpu.sync_copy(dst_hbm.at[w], didx_v)
        plsc.subcore_barrier()

        for b in range(NBUF):
            pltpu.async_copy(g_hbm.at[sidx_v.at[b]], rows_v.at[b], sems[b])

        @pl.loop(0, ngroups - 1)
        def _(g):
            j = g * NBUF
            for b in range(NBUF):
                pltpu.make_async_copy(
                    g_hbm.at[sidx_v.at[b]], rows_v.at[b], sems[b]).wait()
                pltpu.sync_copy(rows_v.at[b], agg_sh.at[didx_v.at[j + b]],
                                add=True)
                pltpu.async_copy(
                    g_hbm.at[sidx_v.at[j + NBUF + b]], rows_v.at[b], sems[b])

        jl = (ngroups - 1) * NBUF
        for b in range(NBUF):
            pltpu.make_async_copy(
                g_hbm.at[sidx_v.at[b]], rows_v.at[b], sems[b]).wait()
            pltpu.sync_copy(rows_v.at[b], agg_sh.at[didx_v.at[jl + b]],
                            add=True)

        plsc.subcore_barrier()
        pltpu.sync_copy(agg_sh.at[pl.ds(base, rps)],
                        out_hbm.at[c, pl.ds(base, rps)])

    return agg_kernel


def _g1_body(x_ref, w1_ref, degp_ref, g1_ref, dinv_ref):
    deg = degp_ref[0, :] + degp_ref[1, :] + 1.0
    dinv = lax.rsqrt(deg)
    h = jnp.dot(x_ref[...], w1_ref[...], preferred_element_type=jnp.float32)
    g1_ref[...] = h * dinv[:, None]
    dinv_ref[...] = dinv


def _g2_body(p_ref, g1_ref, dinv_ref, b1_ref, g2_ref):
    dinv = dinv_ref[...][:, None]
    agg = p_ref[0] + p_ref[1] + g1_ref[...]
    out1 = agg * dinv + b1_ref[...][None, :]
    g2_ref[...] = jnp.maximum(out1, 0.0) * dinv


def _out_body(p_ref, g2_ref, dinv_ref, w2_ref, b2_ref, o_ref):
    dinv = dinv_ref[...][:, None]
    agg = (p_ref[0] + p_ref[1] + g2_ref[...]) * dinv
    v = jnp.dot(agg, w2_ref[...], preferred_element_type=jnp.float32)
    v = v + b2_ref[...][None, :]
    m = jnp.max(v, axis=1, keepdims=True)
    lse = m + jnp.log(jnp.sum(jnp.exp(v - m), axis=1, keepdims=True))
    o_ref[...] = v - lse


def _g1_call(x_pad, w1, degp):
    n_pad, d = x_pad.shape
    h = w1.shape[1]
    return pl.pallas_call(
        _g1_body,
        grid=(n_pad // BR,),
        in_specs=[
            pl.BlockSpec((BR, d), lambda i: (i, 0)),
            pl.BlockSpec((d, h), lambda i: (0, 0)),
            pl.BlockSpec((NC, BR), lambda i: (0, i)),
        ],
        out_specs=[
            pl.BlockSpec((BR, h), lambda i: (i, 0)),
            pl.BlockSpec((BR,), lambda i: (i,)),
        ],
        out_shape=[
            jax.ShapeDtypeStruct((n_pad, h), jnp.float32),
            jax.ShapeDtypeStruct((n_pad,), jnp.float32),
        ],
    )(x_pad, w1, degp)


def _g2_call(p, g1, dinv, b1):
    n_pad, h = g1.shape
    return pl.pallas_call(
        _g2_body,
        grid=(n_pad // BR,),
        in_specs=[
            pl.BlockSpec((NC, BR, h), lambda i: (0, i, 0)),
            pl.BlockSpec((BR, h), lambda i: (i, 0)),
            pl.BlockSpec((BR,), lambda i: (i,)),
            pl.BlockSpec((h,), lambda i: (0,)),
        ],
        out_specs=pl.BlockSpec((BR, h), lambda i: (i, 0)),
        out_shape=jax.ShapeDtypeStruct((n_pad, h), jnp.float32),
    )(p, g1, dinv, b1)


def _out_call(p, g2, dinv, w2, b2):
    n_pad, h = g2.shape
    c = w2.shape[1]
    return pl.pallas_call(
        _out_body,
        grid=(n_pad // BR,),
        in_specs=[
            pl.BlockSpec((NC, BR, h), lambda i: (0, i, 0)),
            pl.BlockSpec((BR, h), lambda i: (i, 0)),
            pl.BlockSpec((BR,), lambda i: (i,)),
            pl.BlockSpec((h, c), lambda i: (0, 0)),
            pl.BlockSpec((c,), lambda i: (0,)),
        ],
        out_specs=pl.BlockSpec((BR, c), lambda i: (i, 0)),
        out_shape=jax.ShapeDtypeStruct((n_pad, c), jnp.float32),
    )(p, g2, dinv, w2, b2)


def kernel(x, edge_index, W1, b1, W2, b2):
    n, d = x.shape
    e = edge_index.shape[1]
    n_pad = ((n + 1 + NS * CHUNK - 1) // (NS * CHUNK)) * (NS * CHUNK)
    cpw = (e + NW * CHUNK - 1) // (NW * CHUNK)
    cpw = ((cpw + NBUF - 1) // NBUF) * NBUF
    e_pad = NW * cpw * CHUNK

    pad = jnp.full((e_pad - e,), n, dtype=jnp.int32)
    src_r = jnp.concatenate([edge_index[0], pad]).reshape(NW, cpw, CHUNK)
    dst_r = jnp.concatenate([edge_index[1], pad]).reshape(NW, cpw, CHUNK)
    x_pad = jnp.pad(x, ((0, n_pad - n), (0, 0)))

    agg = _make_agg_kernel(n_pad, cpw)
    degp = _make_deg_kernel(n_pad, cpw)(dst_r)

    g1, dinv = _g1_call(x_pad, W1, degp)
    p1 = agg(g1, src_r, dst_r)
    g2 = _g2_call(p1, g1, dinv, b1)
    p2 = agg(g2, src_r, dst_r)
    out = _out_call(p2, g2, dinv, W2, b2)
    return out[:n]


# speedup vs baseline: 35.5696x; 1.0073x over previous
import functools

import jax
import jax.numpy as jnp
from jax import lax
from jax.experimental import pallas as pl
from jax.experimental.pallas import tpu as pltpu
from jax.experimental.pallas import tpu_sc as plsc

NC = 2
NS = 16
NW = NC * NS
LANES = 16
CHUNK = 128
NBUF = 8
BR = 256


def _sc_mesh():
    return plsc.VectorSubcoreMesh(core_axis_name="c", subcore_axis_name="s")


def _make_deg_kernel(n_pad, cpw):
    rps = n_pad // NS

    @functools.partial(
        pl.kernel,
        out_type=jax.ShapeDtypeStruct((NC, n_pad), jnp.float32),
        mesh=_sc_mesh(),
        scratch_types=[
            pltpu.VMEM((cpw, CHUNK), jnp.int32),
            pltpu.VMEM((CHUNK,), jnp.float32),
            pltpu.VMEM_SHARED((n_pad,), jnp.float32),
            pltpu.SemaphoreType.DMA,
        ],
    )
    def deg_kernel(dst_hbm, out_hbm, didx_v, ones_v, deg_sh, sem):
        c = lax.axis_index("c")
        s = lax.axis_index("s")
        w = c * NS + s
        base = pl.multiple_of(s * rps, 8)

        @pl.loop(0, CHUNK, step=LANES)
        def _(i):
            ones_v[pl.ds(i, LANES)] = jnp.zeros((LANES,), jnp.float32)

        for k in range(rps // CHUNK):
            pltpu.sync_copy(ones_v, deg_sh.at[pl.ds(base + k * CHUNK, CHUNK)])

        @pl.loop(0, CHUNK, step=LANES)
        def _(i):
            ones_v[pl.ds(i, LANES)] = jnp.ones((LANES,), jnp.float32)

        pltpu.sync_copy(dst_hbm.at[w], didx_v)
        plsc.subcore_barrier()

        @pl.loop(0, cpw)
        def _(j):
            pltpu.sync_copy(ones_v, deg_sh.at[didx_v.at[j]], add=True)

        plsc.subcore_barrier()
        pltpu.sync_copy(deg_sh.at[pl.ds(base, rps)],
                        out_hbm.at[c, pl.ds(base, rps)])

    return deg_kernel


def _make_agg_kernel(n_pad, cpw):
    rps = n_pad // NS
    assert cpw % NBUF == 0 and cpw // NBUF >= 2
    ngroups = cpw // NBUF

    @functools.partial(
        pl.kernel,
        out_type=jax.ShapeDtypeStruct((NC, n_pad, LANES), jnp.float32),
        mesh=_sc_mesh(),
        scratch_types=[
            pltpu.VMEM((cpw, CHUNK), jnp.int32),
            pltpu.VMEM((cpw, CHUNK), jnp.int32),
            pltpu.VMEM((NBUF, CHUNK, LANES), jnp.float32),
            pltpu.VMEM_SHARED((n_pad, LANES), jnp.float32),
        ] + [pltpu.SemaphoreType.DMA] * NBUF,
        compiler_params=pltpu.CompilerParams(use_tc_tiling_on_sc=False),
    )
    def agg_kernel(g_hbm, src_hbm, dst_hbm, out_hbm,
                   sidx_v, didx_v, rows_v, agg_sh, *sems):
        c = lax.axis_index("c")
        s = lax.axis_index("s")
        w = c * NS + s
        base = pl.multiple_of(s * rps, 8)

        @pl.loop(0, CHUNK)
        def _(i):
            rows_v[0, i, :] = jnp.zeros((LANES,), jnp.float32)

        for k in range(rps // CHUNK):
            pltpu.sync_copy(rows_v.at[0], agg_sh.at[pl.ds(base + k * CHUNK, CHUNK)])

        pltpu.sync_copy(src_hbm.at[w], sidx_v)
        pltpu.sync_copy(dst_hbm.at[w], didx_v)
        plsc.subcore_barrier()

        for b in range(NBUF):
            pltpu.async_copy(g_hbm.at[sidx_v.at[b]], rows_v.at[b], sems[b])

        @pl.loop(0, ngroups - 1)
        def _(g):
            j = g * NBUF
            for b in range(NBUF):
                pltpu.make_async_copy(
                    g_hbm.at[sidx_v.at[b]], rows_v.at[b], sems[b]).wait()
                pltpu.sync_copy(rows_v.at[b], agg_sh.at[didx_v.at[j + b]],
                                add=True)
                pltpu.async_copy(
                    g_hbm.at[sidx_v.at[j + NBUF + b]], rows_v.at[b], sems[b])

        jl = (ngroups - 1) * NBUF
        for b in range(NBUF):
            pltpu.make_async_copy(
                g_hbm.at[sidx_v.at[b]], rows_v.at[b], sems[b]).wait()
            pltpu.sync_copy(rows_v.at[b], agg_sh.at[didx_v.at[jl + b]],
                            add=True)

        plsc.subcore_barrier()
        pltpu.sync_copy(agg_sh.at[pl.ds(base, rps)],
                        out_hbm.at[c, pl.ds(base, rps)])

    return agg_kernel


def _g1_body(x_ref, w1_ref, degp_ref, g1_ref, dinv_ref):
    deg = degp_ref[0, :] + degp_ref[1, :] + 1.0
    dinv = lax.rsqrt(deg)
    h = jnp.dot(x_ref[...], w1_ref[...], preferred_element_type=jnp.float32)
    g1_ref[...] = h * dinv[:, None]
    dinv_ref[...] = dinv


def _g2_body(p_ref, g1_ref, dinv_ref, b1_ref, g2_ref):
    dinv = dinv_ref[...][:, None]
    agg = p_ref[0] + p_ref[1] + g1_ref[...]
    out1 = agg * dinv + b1_ref[...][None, :]
    g2_ref[...] = jnp.maximum(out1, 0.0) * dinv


def _out_body(p_ref, g2_ref, dinv_ref, w2_ref, b2_ref, o_ref):
    dinv = dinv_ref[...][:, None]
    agg = (p_ref[0] + p_ref[1] + g2_ref[...]) * dinv
    v = jnp.dot(agg, w2_ref[...], preferred_element_type=jnp.float32)
    v = v + b2_ref[...][None, :]
    m = jnp.max(v, axis=1, keepdims=True)
    lse = m + jnp.log(jnp.sum(jnp.exp(v - m), axis=1, keepdims=True))
    o_ref[...] = v - lse


def _g1_call(x_pad, w1, degp):
    n_pad, d = x_pad.shape
    h = w1.shape[1]
    return pl.pallas_call(
        _g1_body,
        grid=(n_pad // BR,),
        in_specs=[
            pl.BlockSpec((BR, d), lambda i: (i, 0)),
            pl.BlockSpec((d, h), lambda i: (0, 0)),
            pl.BlockSpec((NC, BR), lambda i: (0, i)),
        ],
        out_specs=[
            pl.BlockSpec((BR, h), lambda i: (i, 0)),
            pl.BlockSpec((BR,), lambda i: (i,)),
        ],
        out_shape=[
            jax.ShapeDtypeStruct((n_pad, h), jnp.float32),
            jax.ShapeDtypeStruct((n_pad,), jnp.float32),
        ],
    )(x_pad, w1, degp)


def _g2_call(p, g1, dinv, b1):
    n_pad, h = g1.shape
    return pl.pallas_call(
        _g2_body,
        grid=(n_pad // BR,),
        in_specs=[
            pl.BlockSpec((NC, BR, h), lambda i: (0, i, 0)),
            pl.BlockSpec((BR, h), lambda i: (i, 0)),
            pl.BlockSpec((BR,), lambda i: (i,)),
            pl.BlockSpec((h,), lambda i: (0,)),
        ],
        out_specs=pl.BlockSpec((BR, h), lambda i: (i, 0)),
        out_shape=jax.ShapeDtypeStruct((n_pad, h), jnp.float32),
    )(p, g1, dinv, b1)


def _out_call(p, g2, dinv, w2, b2):
    n_pad, h = g2.shape
    c = w2.shape[1]
    return pl.pallas_call(
        _out_body,
        grid=(n_pad // BR,),
        in_specs=[
            pl.BlockSpec((NC, BR, h), lambda i: (0, i, 0)),
            pl.BlockSpec((BR, h), lambda i: (i, 0)),
            pl.BlockSpec((BR,), lambda i: (i,)),
            pl.BlockSpec((h, c), lambda i: (0, 0)),
            pl.BlockSpec((c,), lambda i: (0,)),
        ],
        out_specs=pl.BlockSpec((BR, c), lambda i: (i, 0)),
        out_shape=jax.ShapeDtypeStruct((n_pad, c), jnp.float32),
    )(p, g2, dinv, w2, b2)


def kernel(x, edge_index, W1, b1, W2, b2):
    n, d = x.shape
    e = edge_index.shape[1]
    n_pad = ((n + 1 + NS * CHUNK - 1) // (NS * CHUNK)) * (NS * CHUNK)
    cpw = (e + NW * CHUNK - 1) // (NW * CHUNK)
    cpw = ((cpw + NBUF - 1) // NBUF) * NBUF
    e_pad = NW * cpw * CHUNK

    pad = jnp.full((e_pad - e,), n, dtype=jnp.int32)
    src_r = jnp.concatenate([edge_index[0], pad]).reshape(NW, cpw, CHUNK)
    dst_r = jnp.concatenate([edge_index[1], pad]).reshape(NW, cpw, CHUNK)
    x_pad = jnp.pad(x, ((0, n_pad - n), (0, 0)))

    agg = _make_agg_kernel(n_pad, cpw)
    degp = _make_deg_kernel(n_pad, cpw)(dst_r)

    g1, dinv = _g1_call(x_pad, W1, degp)
    p1 = agg(g1, src_r, dst_r)
    g2 = _g2_call(p1, g1, dinv, b1)
    p2 = agg(g2, src_r, dst_r)
    out = _out_call(p2, g2, dinv, W2, b2)
    return out[:n]


# speedup vs baseline: 47.1441x; 1.3254x over previous
import functools

import jax
import jax.numpy as jnp
from jax import lax
from jax.experimental import pallas as pl
from jax.experimental.pallas import tpu as pltpu
from jax.experimental.pallas import tpu_sc as plsc

NC = 2
NS = 16
NW = NC * NS
LANES = 16
CHUNK = 128
NBUF = 8
BR = 2048


def _sc_mesh():
    return plsc.VectorSubcoreMesh(core_axis_name="c", subcore_axis_name="s")


def _make_deg_kernel(n_pad, cpw):
    rps = n_pad // NS

    @functools.partial(
        pl.kernel,
        out_type=jax.ShapeDtypeStruct((NC, n_pad), jnp.float32),
        mesh=_sc_mesh(),
        scratch_types=[
            pltpu.VMEM((cpw, CHUNK), jnp.int32),
            pltpu.VMEM((CHUNK,), jnp.float32),
            pltpu.VMEM_SHARED((n_pad,), jnp.float32),
            pltpu.SemaphoreType.DMA,
        ],
    )
    def deg_kernel(dst_hbm, out_hbm, didx_v, ones_v, deg_sh, sem):
        c = lax.axis_index("c")
        s = lax.axis_index("s")
        w = c * NS + s
        base = pl.multiple_of(s * rps, 8)

        @pl.loop(0, CHUNK, step=LANES)
        def _(i):
            ones_v[pl.ds(i, LANES)] = jnp.zeros((LANES,), jnp.float32)

        for k in range(rps // CHUNK):
            pltpu.sync_copy(ones_v, deg_sh.at[pl.ds(base + k * CHUNK, CHUNK)])

        @pl.loop(0, CHUNK, step=LANES)
        def _(i):
            ones_v[pl.ds(i, LANES)] = jnp.ones((LANES,), jnp.float32)

        pltpu.sync_copy(dst_hbm.at[w], didx_v)
        plsc.subcore_barrier()

        @pl.loop(0, cpw)
        def _(j):
            pltpu.sync_copy(ones_v, deg_sh.at[didx_v.at[j]], add=True)

        plsc.subcore_barrier()
        pltpu.sync_copy(deg_sh.at[pl.ds(base, rps)],
                        out_hbm.at[c, pl.ds(base, rps)])

    return deg_kernel


def _make_agg_kernel(n_pad, cpw):
    rps = n_pad // NS
    assert cpw % NBUF == 0 and cpw // NBUF >= 2
    ngroups = cpw // NBUF

    @functools.partial(
        pl.kernel,
        out_type=jax.ShapeDtypeStruct((NC, n_pad, LANES), jnp.float32),
        mesh=_sc_mesh(),
        scratch_types=[
            pltpu.VMEM((cpw, CHUNK), jnp.int32),
            pltpu.VMEM((cpw, CHUNK), jnp.int32),
            pltpu.VMEM((NBUF, CHUNK, LANES), jnp.float32),
            pltpu.VMEM_SHARED((n_pad, LANES), jnp.float32),
        ] + [pltpu.SemaphoreType.DMA] * NBUF,
        compiler_params=pltpu.CompilerParams(use_tc_tiling_on_sc=False),
    )
    def agg_kernel(g_hbm, src_hbm, dst_hbm, out_hbm,
                   sidx_v, didx_v, rows_v, agg_sh, *sems):
        c = lax.axis_index("c")
        s = lax.axis_index("s")
        w = c * NS + s
        base = pl.multiple_of(s * rps, 8)

        @pl.loop(0, CHUNK)
        def _(i):
            rows_v[0, i, :] = jnp.zeros((LANES,), jnp.float32)

        for k in range(rps // CHUNK):
            pltpu.sync_copy(rows_v.at[0], agg_sh.at[pl.ds(base + k * CHUNK, CHUNK)])

        pltpu.sync_copy(src_hbm.at[w], sidx_v)
        pltpu.sync_copy(dst_hbm.at[w], didx_v)
        plsc.subcore_barrier()

        for b in range(NBUF):
            pltpu.async_copy(g_hbm.at[sidx_v.at[b]], rows_v.at[b], sems[b])

        @pl.loop(0, ngroups - 1)
        def _(g):
            j = g * NBUF
            for b in range(NBUF):
                pltpu.make_async_copy(
                    g_hbm.at[sidx_v.at[b]], rows_v.at[b], sems[b]).wait()
                pltpu.sync_copy(rows_v.at[b], agg_sh.at[didx_v.at[j + b]],
                                add=True)
                pltpu.async_copy(
                    g_hbm.at[sidx_v.at[j + NBUF + b]], rows_v.at[b], sems[b])

        jl = (ngroups - 1) * NBUF
        for b in range(NBUF):
            pltpu.make_async_copy(
                g_hbm.at[sidx_v.at[b]], rows_v.at[b], sems[b]).wait()
            pltpu.sync_copy(rows_v.at[b], agg_sh.at[didx_v.at[jl + b]],
                            add=True)

        plsc.subcore_barrier()
        pltpu.sync_copy(agg_sh.at[pl.ds(base, rps)],
                        out_hbm.at[c, pl.ds(base, rps)])

    return agg_kernel


def _g1_body(x_ref, w1_ref, degp_ref, g1_ref, dinv_ref):
    deg = degp_ref[0, :] + degp_ref[1, :] + 1.0
    dinv = lax.rsqrt(deg)
    h = jnp.dot(x_ref[...], w1_ref[...], preferred_element_type=jnp.float32)
    g1_ref[...] = h * dinv[:, None]
    dinv_ref[...] = dinv


def _g2_body(p_ref, g1_ref, dinv_ref, b1_ref, g2_ref):
    dinv = dinv_ref[...][:, None]
    agg = p_ref[0] + p_ref[1] + g1_ref[...]
    out1 = agg * dinv + b1_ref[...][None, :]
    g2_ref[...] = jnp.maximum(out1, 0.0) * dinv


def _out_body(p_ref, g2_ref, dinv_ref, w2_ref, b2_ref, o_ref):
    dinv = dinv_ref[...][:, None]
    agg = (p_ref[0] + p_ref[1] + g2_ref[...]) * dinv
    v = jnp.dot(agg, w2_ref[...], preferred_element_type=jnp.float32)
    v = v + b2_ref[...][None, :]
    m = jnp.max(v, axis=1, keepdims=True)
    lse = m + jnp.log(jnp.sum(jnp.exp(v - m), axis=1, keepdims=True))
    o_ref[...] = v - lse


def _g1_call(x_pad, w1, degp):
    n_pad, d = x_pad.shape
    h = w1.shape[1]
    return pl.pallas_call(
        _g1_body,
        grid=(n_pad // BR,),
        in_specs=[
            pl.BlockSpec((BR, d), lambda i: (i, 0)),
            pl.BlockSpec((d, h), lambda i: (0, 0)),
            pl.BlockSpec((NC, BR), lambda i: (0, i)),
        ],
        out_specs=[
            pl.BlockSpec((BR, h), lambda i: (i, 0)),
            pl.BlockSpec((BR,), lambda i: (i,)),
        ],
        out_shape=[
            jax.ShapeDtypeStruct((n_pad, h), jnp.float32),
            jax.ShapeDtypeStruct((n_pad,), jnp.float32),
        ],
    )(x_pad, w1, degp)


def _g2_call(p, g1, dinv, b1):
    n_pad, h = g1.shape
    return pl.pallas_call(
        _g2_body,
        grid=(n_pad // BR,),
        in_specs=[
            pl.BlockSpec((NC, BR, h), lambda i: (0, i, 0)),
            pl.BlockSpec((BR, h), lambda i: (i, 0)),
            pl.BlockSpec((BR,), lambda i: (i,)),
            pl.BlockSpec((h,), lambda i: (0,)),
        ],
        out_specs=pl.BlockSpec((BR, h), lambda i: (i, 0)),
        out_shape=jax.ShapeDtypeStruct((n_pad, h), jnp.float32),
    )(p, g1, dinv, b1)


def _out_call(p, g2, dinv, w2, b2):
    n_pad, h = g2.shape
    c = w2.shape[1]
    return pl.pallas_call(
        _out_body,
        grid=(n_pad // BR,),
        in_specs=[
            pl.BlockSpec((NC, BR, h), lambda i: (0, i, 0)),
            pl.BlockSpec((BR, h), lambda i: (i, 0)),
            pl.BlockSpec((BR,), lambda i: (i,)),
            pl.BlockSpec((h, c), lambda i: (0, 0)),
            pl.BlockSpec((c,), lambda i: (0,)),
        ],
        out_specs=pl.BlockSpec((BR, c), lambda i: (i, 0)),
        out_shape=jax.ShapeDtypeStruct((n_pad, c), jnp.float32),
    )(p, g2, dinv, w2, b2)


def kernel(x, edge_index, W1, b1, W2, b2):
    n, d = x.shape
    e = edge_index.shape[1]
    n_pad = ((n + 1 + NS * CHUNK - 1) // (NS * CHUNK)) * (NS * CHUNK)
    cpw = (e + NW * CHUNK - 1) // (NW * CHUNK)
    cpw = ((cpw + NBUF - 1) // NBUF) * NBUF
    e_pad = NW * cpw * CHUNK

    ei_pad = jnp.pad(edge_index, ((0, 0), (0, e_pad - e)), constant_values=n)
    src_r = ei_pad[0].reshape(NW, cpw, CHUNK)
    dst_r = ei_pad[1].reshape(NW, cpw, CHUNK)
    x_pad = jnp.pad(x, ((0, n_pad - n), (0, 0)))

    agg = _make_agg_kernel(n_pad, cpw)
    degp = _make_deg_kernel(n_pad, cpw)(dst_r)

    g1, dinv = _g1_call(x_pad, W1, degp)
    p1 = agg(g1, src_r, dst_r)
    g2 = _g2_call(p1, g1, dinv, b1)
    p2 = agg(g2, src_r, dst_r)
    out = _out_call(p2, g2, dinv, W2, b2)
    return out[:n]


# speedup vs baseline: 51.0159x; 1.0821x over previous
import functools

import jax
import jax.numpy as jnp
from jax import lax
from jax.experimental import pallas as pl
from jax.experimental.pallas import tpu as pltpu
from jax.experimental.pallas import tpu_sc as plsc

NC = 2
NS = 16
NW = NC * NS
LANES = 16
CHUNK = 128
NBUF = 8
BR = 2048


def _sc_mesh():
    return plsc.VectorSubcoreMesh(core_axis_name="c", subcore_axis_name="s")


def _make_deg_kernel(n_pad, cpw):
    rps = n_pad // NS

    @functools.partial(
        pl.kernel,
        out_type=jax.ShapeDtypeStruct((NC, n_pad, LANES), jnp.float32),
        mesh=_sc_mesh(),
        scratch_types=[
            pltpu.VMEM((cpw, CHUNK), jnp.int32),
            pltpu.VMEM((CHUNK, LANES), jnp.float32),
            pltpu.VMEM_SHARED((n_pad, LANES), jnp.float32),
            pltpu.SemaphoreType.DMA,
        ],
    )
    def deg_kernel(dst_hbm, out_hbm, didx_v, ones_v, deg_sh, sem):
        c = lax.axis_index("c")
        s = lax.axis_index("s")
        w = c * NS + s
        base = pl.multiple_of(s * rps, 8)

        @pl.loop(0, CHUNK)
        def _(i):
            ones_v[i, :] = jnp.zeros((LANES,), jnp.float32)

        for k in range(rps // CHUNK):
            pltpu.sync_copy(ones_v, deg_sh.at[pl.ds(base + k * CHUNK, CHUNK)])

        @pl.loop(0, CHUNK)
        def _(i):
            ones_v[i, :] = jnp.ones((LANES,), jnp.float32)

        pltpu.sync_copy(dst_hbm.at[w], didx_v)
        plsc.subcore_barrier()

        @pl.loop(0, cpw)
        def _(j):
            pltpu.sync_copy(ones_v, deg_sh.at[didx_v.at[j]], add=True)

        plsc.subcore_barrier()
        pltpu.sync_copy(deg_sh.at[pl.ds(base, rps)],
                        out_hbm.at[c, pl.ds(base, rps)])

    return deg_kernel


def _make_agg_kernel(n_pad, cpw):
    rps = n_pad // NS
    assert cpw % NBUF == 0 and cpw // NBUF >= 2
    ngroups = cpw // NBUF

    @functools.partial(
        pl.kernel,
        out_type=jax.ShapeDtypeStruct((NC, n_pad, LANES), jnp.float32),
        mesh=_sc_mesh(),
        scratch_types=[
            pltpu.VMEM((cpw, CHUNK), jnp.int32),
            pltpu.VMEM((cpw, CHUNK), jnp.int32),
            pltpu.VMEM((NBUF, CHUNK, LANES), jnp.float32),
            pltpu.VMEM_SHARED((n_pad, LANES), jnp.float32),
        ] + [pltpu.SemaphoreType.DMA] * NBUF,
        compiler_params=pltpu.CompilerParams(use_tc_tiling_on_sc=False),
    )
    def agg_kernel(g_hbm, src_hbm, dst_hbm, out_hbm,
                   sidx_v, didx_v, rows_v, agg_sh, *sems):
        c = lax.axis_index("c")
        s = lax.axis_index("s")
        w = c * NS + s
        base = pl.multiple_of(s * rps, 8)

        @pl.loop(0, CHUNK)
        def _(i):
            rows_v[0, i, :] = jnp.zeros((LANES,), jnp.float32)

        for k in range(rps // CHUNK):
            pltpu.sync_copy(rows_v.at[0], agg_sh.at[pl.ds(base + k * CHUNK, CHUNK)])

        pltpu.sync_copy(src_hbm.at[w], sidx_v)
        pltpu.sync_copy(dst_hbm.at[w], didx_v)
        plsc.subcore_barrier()

        for b in range(NBUF):
            pltpu.async_copy(g_hbm.at[sidx_v.at[b]], rows_v.at[b], sems[b])

        @pl.loop(0, ngroups - 1)
        def _(g):
            j = g * NBUF
            for b in range(NBUF):
                pltpu.make_async_copy(
                    g_hbm.at[sidx_v.at[b]], rows_v.at[b], sems[b]).wait()
                pltpu.sync_copy(rows_v.at[b], agg_sh.at[didx_v.at[j + b]],
                                add=True)
                pltpu.async_copy(
                    g_hbm.at[sidx_v.at[j + NBUF + b]], rows_v.at[b], sems[b])

        jl = (ngroups - 1) * NBUF
        for b in range(NBUF):
            pltpu.make_async_copy(
                g_hbm.at[sidx_v.at[b]], rows_v.at[b], sems[b]).wait()
            pltpu.sync_copy(rows_v.at[b], agg_sh.at[didx_v.at[jl + b]],
                            add=True)

        plsc.subcore_barrier()
        pltpu.sync_copy(agg_sh.at[pl.ds(base, rps)],
                        out_hbm.at[c, pl.ds(base, rps)])

    return agg_kernel


def _g1_body(xr_ref, w1b_ref, degp_ref, g1_ref, dinv_ref):
    deg = degp_ref[0] + degp_ref[1] + 1.0
    dinv = lax.rsqrt(deg)
    h = jnp.dot(xr_ref[...], w1b_ref[...], preferred_element_type=jnp.float32)
    g1_ref[...] = h * dinv
    dinv_ref[...] = dinv


def _g2_body(zero_row, p_ref, g1_ref, dinv_ref, b1t_ref, g2_ref):
    dinv = dinv_ref[...]
    agg = p_ref[0] + p_ref[1] + g1_ref[...]
    out1 = agg * dinv + b1t_ref[...][None, :]
    g2_ref[...] = jnp.maximum(out1, 0.0) * dinv
    g2_ref[zero_row, 0:LANES] = jnp.zeros((LANES,), jnp.float32)


def _out_body(p_ref, g2_ref, dinv_ref, w2b_ref, pswap_ref, b2t_ref, o_ref):
    agg = (p_ref[0] + p_ref[1] + g2_ref[...]) * dinv_ref[...]
    v = jnp.dot(agg, w2b_ref[...], preferred_element_type=jnp.float32)
    v = v + b2t_ref[...][None, :]
    vs = jnp.dot(v, pswap_ref[...], preferred_element_type=jnp.float32)
    m = jnp.maximum(v, vs)
    lse = m + jnp.log(jnp.exp(v - m) + jnp.exp(vs - m))
    o_ref[...] = v - lse


def _g1_call(x_rs, w1b, degp):
    npk = x_rs.shape[0]
    return pl.pallas_call(
        _g1_body,
        out_shape=[
            jax.ShapeDtypeStruct((npk, 128), jnp.float32),
            jax.ShapeDtypeStruct((npk, 128), jnp.float32),
        ],
    )(x_rs, w1b, degp)


def _g2_call(zero_row, p, g1, dinv, b1t):
    npk = g1.shape[0]
    return pl.pallas_call(
        functools.partial(_g2_body, zero_row),
        out_shape=jax.ShapeDtypeStruct((npk, 128), jnp.float32),
    )(p, g1, dinv, b1t)


def _out_call(p, g2, dinv, w2b, pswap, b2t):
    npk = g2.shape[0]
    return pl.pallas_call(
        _out_body,
        out_shape=jax.ShapeDtypeStruct((npk, 2 * 8), jnp.float32),
    )(p, g2, dinv, w2b, pswap, b2t)


def kernel(x, edge_index, W1, b1, W2, b2):
    n, d = x.shape
    e = edge_index.shape[1]
    h = W1.shape[1]
    c = W2.shape[1]
    n_pad = ((n + 1 + NS * CHUNK - 1) // (NS * CHUNK)) * (NS * CHUNK)
    cpw = (e + NW * CHUNK - 1) // (NW * CHUNK)
    cpw = ((cpw + NBUF - 1) // NBUF) * NBUF
    e_pad = NW * cpw * CHUNK
    npk = n_pad // 8

    ei_pad = jnp.pad(edge_index, ((0, 0), (0, e_pad - e)), constant_values=n)
    src_r = ei_pad[0].reshape(NW, cpw, CHUNK)
    dst_r = ei_pad[1].reshape(NW, cpw, CHUNK)
    x_rs = jnp.pad(x, ((0, n_pad - n), (0, 0))).reshape(npk, 8 * d)

    eye8 = jnp.eye(8, dtype=jnp.float32)
    w1b = jnp.kron(eye8, W1)
    w2b = jnp.kron(eye8, W2)
    pswap = jnp.kron(eye8, jnp.ones((c, c), jnp.float32)
                     - jnp.eye(c, dtype=jnp.float32))
    b1t = jnp.tile(b1, 8)
    b2t = jnp.tile(b2, 8)

    agg = _make_agg_kernel(n_pad, cpw)
    degp = _make_deg_kernel(n_pad, cpw)(dst_r)
    degp_pk = degp.reshape(NC, npk, 128)

    g1, dinv = _g1_call(x_rs, w1b, degp_pk)
    p1 = agg(g1.reshape(n_pad, LANES), src_r, dst_r)
    g2 = _g2_call(n // 8, p1.reshape(NC, npk, 128), g1, dinv, b1t)
    p2 = agg(g2.reshape(n_pad, LANES), src_r, dst_r)
    out = _out_call(p2.reshape(NC, npk, 128), g2, dinv, w2b, pswap, b2t)
    return out.reshape(n_pad, c)[:n]


# speedup vs baseline: 51.1821x; 1.0033x over previous
import functools

import jax
import jax.numpy as jnp
from jax import lax
from jax.experimental import pallas as pl
from jax.experimental.pallas import tpu as pltpu
from jax.experimental.pallas import tpu_sc as plsc

NC = 2
NS = 16
NW = NC * NS
LANES = 16
CHUNK = 128
NBUF = 8
BR = 2048


def _sc_mesh():
    return plsc.VectorSubcoreMesh(core_axis_name="c", subcore_axis_name="s")


def _make_deg_kernel(n_pad, cpw):
    rps = n_pad // NS

    @functools.partial(
        pl.kernel,
        out_type=jax.ShapeDtypeStruct((NC, n_pad), jnp.float32),
        mesh=_sc_mesh(),
        scratch_types=[
            pltpu.VMEM((cpw, CHUNK), jnp.int32),
            pltpu.VMEM((CHUNK,), jnp.float32),
            pltpu.VMEM_SHARED((n_pad,), jnp.float32),
            pltpu.SemaphoreType.DMA,
        ],
    )
    def deg_kernel(dst_hbm, out_hbm, didx_v, ones_v, deg_sh, sem):
        c = lax.axis_index("c")
        s = lax.axis_index("s")
        w = c * NS + s
        base = pl.multiple_of(s * rps, 8)

        @pl.loop(0, CHUNK, step=LANES)
        def _(i):
            ones_v[pl.ds(i, LANES)] = jnp.zeros((LANES,), jnp.float32)

        for k in range(rps // CHUNK):
            pltpu.sync_copy(ones_v, deg_sh.at[pl.ds(base + k * CHUNK, CHUNK)])

        @pl.loop(0, CHUNK, step=LANES)
        def _(i):
            ones_v[pl.ds(i, LANES)] = jnp.ones((LANES,), jnp.float32)

        pltpu.sync_copy(dst_hbm.at[w], didx_v)
        plsc.subcore_barrier()

        @pl.loop(0, cpw)
        def _(j):
            pltpu.sync_copy(ones_v, deg_sh.at[didx_v.at[j]], add=True)

        plsc.subcore_barrier()
        pltpu.sync_copy(deg_sh.at[pl.ds(base, rps)],
                        out_hbm.at[c, pl.ds(base, rps)])

    return deg_kernel


def _make_agg_kernel(n_pad, cpw):
    rps = n_pad // NS
    assert cpw % NBUF == 0 and cpw // NBUF >= 2
    ngroups = cpw // NBUF

    @functools.partial(
        pl.kernel,
        out_type=jax.ShapeDtypeStruct((NC, n_pad, LANES), jnp.float32),
        mesh=_sc_mesh(),
        scratch_types=[
            pltpu.VMEM((cpw, CHUNK), jnp.int32),
            pltpu.VMEM((cpw, CHUNK), jnp.int32),
            pltpu.VMEM((NBUF, CHUNK, LANES), jnp.float32),
            pltpu.VMEM_SHARED((n_pad, LANES), jnp.float32),
        ] + [pltpu.SemaphoreType.DMA] * NBUF,
        compiler_params=pltpu.CompilerParams(use_tc_tiling_on_sc=False),
    )
    def agg_kernel(g_hbm, src_hbm, dst_hbm, out_hbm,
                   sidx_v, didx_v, rows_v, agg_sh, *sems):
        c = lax.axis_index("c")
        s = lax.axis_index("s")
        w = c * NS + s
        base = pl.multiple_of(s * rps, 8)

        @pl.loop(0, CHUNK)
        def _(i):
            rows_v[0, i, :] = jnp.zeros((LANES,), jnp.float32)

        for k in range(rps // CHUNK):
            pltpu.sync_copy(rows_v.at[0], agg_sh.at[pl.ds(base + k * CHUNK, CHUNK)])

        pltpu.sync_copy(src_hbm.at[w], sidx_v)
        pltpu.sync_copy(dst_hbm.at[w], didx_v)
        plsc.subcore_barrier()

        for b in range(NBUF):
            pltpu.async_copy(g_hbm.at[sidx_v.at[b]], rows_v.at[b], sems[b])

        @pl.loop(0, ngroups - 1)
        def _(g):
            j = g * NBUF
            for b in range(NBUF):
                pltpu.make_async_copy(
                    g_hbm.at[sidx_v.at[b]], rows_v.at[b], sems[b]).wait()
                pltpu.sync_copy(rows_v.at[b], agg_sh.at[didx_v.at[j + b]],
                                add=True)
                pltpu.async_copy(
                    g_hbm.at[sidx_v.at[j + NBUF + b]], rows_v.at[b], sems[b])

        jl = (ngroups - 1) * NBUF
        for b in range(NBUF):
            pltpu.make_async_copy(
                g_hbm.at[sidx_v.at[b]], rows_v.at[b], sems[b]).wait()
            pltpu.sync_copy(rows_v.at[b], agg_sh.at[didx_v.at[jl + b]],
                            add=True)

        plsc.subcore_barrier()
        pltpu.sync_copy(agg_sh.at[pl.ds(base, rps)],
                        out_hbm.at[c, pl.ds(base, rps)])

    return agg_kernel


def _g1_body(xr_ref, w1b_ref, degp_ref, bcast_ref, g1_ref, dinv_ref):
    deg8 = degp_ref[0] + degp_ref[1] + 1.0
    dinv = jnp.dot(lax.rsqrt(deg8), bcast_ref[...],
                   preferred_element_type=jnp.float32)
    h = jnp.dot(xr_ref[...], w1b_ref[...], preferred_element_type=jnp.float32)
    g1_ref[...] = h * dinv
    dinv_ref[...] = dinv


def _g2_body(zero_row, p_ref, g1_ref, dinv_ref, b1t_ref, g2_ref):
    dinv = dinv_ref[...]
    agg = p_ref[0] + p_ref[1] + g1_ref[...]
    out1 = agg * dinv + b1t_ref[...][None, :]
    g2_ref[...] = jnp.maximum(out1, 0.0) * dinv
    g2_ref[zero_row, 0:LANES] = jnp.zeros((LANES,), jnp.float32)


def _out_body(p_ref, g2_ref, dinv_ref, w2b_ref, pswap_ref, b2t_ref, o_ref):
    agg = (p_ref[0] + p_ref[1] + g2_ref[...]) * dinv_ref[...]
    v = jnp.dot(agg, w2b_ref[...], preferred_element_type=jnp.float32)
    v = v + b2t_ref[...][None, :]
    vs = jnp.dot(v, pswap_ref[...], preferred_element_type=jnp.float32)
    m = jnp.maximum(v, vs)
    lse = m + jnp.log(jnp.exp(v - m) + jnp.exp(vs - m))
    o_ref[...] = v - lse


def _g1_call(x_rs, w1b, degp8, bcast):
    npk = x_rs.shape[0]
    return pl.pallas_call(
        _g1_body,
        out_shape=[
            jax.ShapeDtypeStruct((npk, 128), jnp.float32),
            jax.ShapeDtypeStruct((npk, 128), jnp.float32),
        ],
    )(x_rs, w1b, degp8, bcast)


def _g2_call(zero_row, p, g1, dinv, b1t):
    npk = g1.shape[0]
    return pl.pallas_call(
        functools.partial(_g2_body, zero_row),
        out_shape=jax.ShapeDtypeStruct((npk, 128), jnp.float32),
    )(p, g1, dinv, b1t)


def _out_call(p, g2, dinv, w2b, pswap, b2t):
    npk = g2.shape[0]
    return pl.pallas_call(
        _out_body,
        out_shape=jax.ShapeDtypeStruct((npk, 2 * 8), jnp.float32),
    )(p, g2, dinv, w2b, pswap, b2t)


def kernel(x, edge_index, W1, b1, W2, b2):
    n, d = x.shape
    e = edge_index.shape[1]
    h = W1.shape[1]
    c = W2.shape[1]
    n_pad = ((n + 1 + NS * CHUNK - 1) // (NS * CHUNK)) * (NS * CHUNK)
    cpw = (e + NW * CHUNK - 1) // (NW * CHUNK)
    cpw = ((cpw + NBUF - 1) // NBUF) * NBUF
    e_pad = NW * cpw * CHUNK
    npk = n_pad // 8

    ei_pad = jnp.pad(edge_index, ((0, 0), (0, e_pad - e)), constant_values=n)
    src_r = ei_pad[0].reshape(NW, cpw, CHUNK)
    dst_r = ei_pad[1].reshape(NW, cpw, CHUNK)
    x_rs = jnp.pad(x, ((0, n_pad - n), (0, 0))).reshape(npk, 8 * d)

    eye8 = jnp.eye(8, dtype=jnp.float32)
    w1b = jnp.kron(eye8, W1)
    w2b = jnp.kron(eye8, W2)
    pswap = jnp.kron(eye8, jnp.ones((c, c), jnp.float32)
                     - jnp.eye(c, dtype=jnp.float32))
    bcast = jnp.kron(eye8, jnp.ones((1, LANES), jnp.float32))
    b1t = jnp.tile(b1, 8)
    b2t = jnp.tile(b2, 8)

    agg = _make_agg_kernel(n_pad, cpw)
    degp = _make_deg_kernel(n_pad, cpw)(dst_r)
    degp8 = degp.reshape(NC, npk, 8)

    g1, dinv = _g1_call(x_rs, w1b, degp8, bcast)
    p1 = agg(g1.reshape(n_pad, LANES), src_r, dst_r)
    g2 = _g2_call(n // 8, p1.reshape(NC, npk, 128), g1, dinv, b1t)
    p2 = agg(g2.reshape(n_pad, LANES), src_r, dst_r)
    out = _out_call(p2.reshape(NC, npk, 128), g2, dinv, w2b, pswap, b2t)
    return out.reshape(n_pad, c)[:n]


# speedup vs baseline: 52.2662x; 1.0212x over previous
import functools

import jax
import jax.numpy as jnp
from jax import lax
from jax.experimental import pallas as pl
from jax.experimental.pallas import tpu as pltpu
from jax.experimental.pallas import tpu_sc as plsc

NC = 2
NS = 16
NW = NC * NS
LANES = 16
CHUNK = 128
NBUF = 8
BR = 2048


def _sc_mesh():
    return plsc.VectorSubcoreMesh(core_axis_name="c", subcore_axis_name="s")


def _make_deg_kernel(n_pad, cpw):
    rps = n_pad // NS

    @functools.partial(
        pl.kernel,
        out_type=jax.ShapeDtypeStruct((NC, n_pad), jnp.float32),
        mesh=_sc_mesh(),
        scratch_types=[
            pltpu.VMEM((cpw, CHUNK), jnp.int32),
            pltpu.VMEM((CHUNK,), jnp.float32),
            pltpu.VMEM_SHARED((n_pad,), jnp.float32),
            pltpu.SemaphoreType.DMA,
        ],
    )
    def deg_kernel(dst_hbm, out_hbm, didx_v, ones_v, deg_sh, sem):
        c = lax.axis_index("c")
        s = lax.axis_index("s")
        w = c * NS + s
        base = pl.multiple_of(s * rps, 8)

        @pl.loop(0, CHUNK, step=LANES)
        def _(i):
            ones_v[pl.ds(i, LANES)] = jnp.zeros((LANES,), jnp.float32)

        for k in range(rps // CHUNK):
            pltpu.sync_copy(ones_v, deg_sh.at[pl.ds(base + k * CHUNK, CHUNK)])

        @pl.loop(0, CHUNK, step=LANES)
        def _(i):
            ones_v[pl.ds(i, LANES)] = jnp.ones((LANES,), jnp.float32)

        pltpu.sync_copy(dst_hbm.at[w], didx_v)
        plsc.subcore_barrier()

        @pl.loop(0, cpw)
        def _(j):
            pltpu.sync_copy(ones_v, deg_sh.at[didx_v.at[j]], add=True)

        plsc.subcore_barrier()
        pltpu.sync_copy(deg_sh.at[pl.ds(base, rps)],
                        out_hbm.at[c, pl.ds(base, rps)])

    return deg_kernel


def _make_agg_kernel(n_pad, cpw):
    rps = n_pad // NS
    assert cpw % NBUF == 0 and cpw // NBUF >= 2
    ngroups = cpw // NBUF

    @functools.partial(
        pl.kernel,
        out_type=jax.ShapeDtypeStruct((NC, n_pad, LANES), jnp.float32),
        mesh=_sc_mesh(),
        scratch_types=[
            pltpu.VMEM((cpw, CHUNK), jnp.int32),
            pltpu.VMEM((cpw, CHUNK), jnp.int32),
            pltpu.VMEM((NBUF, CHUNK, LANES), jnp.float32),
            pltpu.VMEM_SHARED((n_pad, LANES), jnp.float32),
        ] + [pltpu.SemaphoreType.DMA] * (NBUF + 2),
        compiler_params=pltpu.CompilerParams(use_tc_tiling_on_sc=False),
    )
    def agg_kernel(g_hbm, src_hbm, dst_hbm, out_hbm,
                   sidx_v, didx_v, rows_v, agg_sh, *sems):
        c = lax.axis_index("c")
        s = lax.axis_index("s")
        w = c * NS + s
        base = pl.multiple_of(s * rps, 8)

        ci = pltpu.async_copy(src_hbm.at[w], sidx_v, sems[NBUF])
        cd = pltpu.async_copy(dst_hbm.at[w], didx_v, sems[NBUF + 1])

        @pl.loop(0, CHUNK)
        def _(i):
            rows_v[0, i, :] = jnp.zeros((LANES,), jnp.float32)

        for k in range(rps // CHUNK):
            pltpu.sync_copy(rows_v.at[0], agg_sh.at[pl.ds(base + k * CHUNK, CHUNK)])

        ci.wait()
        cd.wait()
        plsc.subcore_barrier()

        for b in range(NBUF):
            pltpu.async_copy(g_hbm.at[sidx_v.at[b]], rows_v.at[b], sems[b])

        @pl.loop(0, ngroups - 1)
        def _(g):
            j = g * NBUF
            for b in range(NBUF):
                pltpu.make_async_copy(
                    g_hbm.at[sidx_v.at[b]], rows_v.at[b], sems[b]).wait()
                pltpu.sync_copy(rows_v.at[b], agg_sh.at[didx_v.at[j + b]],
                                add=True)
                pltpu.async_copy(
                    g_hbm.at[sidx_v.at[j + NBUF + b]], rows_v.at[b], sems[b])

        jl = (ngroups - 1) * NBUF
        for b in range(NBUF):
            pltpu.make_async_copy(
                g_hbm.at[sidx_v.at[b]], rows_v.at[b], sems[b]).wait()
            pltpu.sync_copy(rows_v.at[b], agg_sh.at[didx_v.at[jl + b]],
                            add=True)

        plsc.subcore_barrier()
        pltpu.sync_copy(agg_sh.at[pl.ds(base, rps)],
                        out_hbm.at[c, pl.ds(base, rps)])

    return agg_kernel


def _g1_body(xr_ref, w1b_ref, degp_ref, bcast_ref, g1_ref, dinv_ref):
    deg8 = degp_ref[0] + degp_ref[1] + 1.0
    dinv = jnp.dot(lax.rsqrt(deg8), bcast_ref[...],
                   preferred_element_type=jnp.float32)
    h = jnp.dot(xr_ref[...], w1b_ref[...], preferred_element_type=jnp.float32)
    g1_ref[...] = h * dinv
    dinv_ref[...] = dinv


def _g2_body(zero_row, p_ref, g1_ref, dinv_ref, b1t_ref, g2_ref):
    dinv = dinv_ref[...]
    agg = p_ref[0] + p_ref[1] + g1_ref[...]
    out1 = agg * dinv + b1t_ref[...][None, :]
    g2_ref[...] = jnp.maximum(out1, 0.0) * dinv
    g2_ref[zero_row, 0:LANES] = jnp.zeros((LANES,), jnp.float32)


def _out_body(p_ref, g2_ref, dinv_ref, w2b_ref, pswap_ref, b2t_ref, o_ref):
    agg = (p_ref[0] + p_ref[1] + g2_ref[...]) * dinv_ref[...]
    v = jnp.dot(agg, w2b_ref[...], preferred_element_type=jnp.float32)
    v = v + b2t_ref[...][None, :]
    vs = jnp.dot(v, pswap_ref[...], preferred_element_type=jnp.float32)
    m = jnp.maximum(v, vs)
    lse = m + jnp.log(jnp.exp(v - m) + jnp.exp(vs - m))
    o_ref[...] = v - lse


def _g1_call(x_rs, w1b, degp8, bcast):
    npk = x_rs.shape[0]
    return pl.pallas_call(
        _g1_body,
        out_shape=[
            jax.ShapeDtypeStruct((npk, 128), jnp.float32),
            jax.ShapeDtypeStruct((npk, 128), jnp.float32),
        ],
    )(x_rs, w1b, degp8, bcast)


def _g2_call(zero_row, p, g1, dinv, b1t):
    npk = g1.shape[0]
    return pl.pallas_call(
        functools.partial(_g2_body, zero_row),
        out_shape=jax.ShapeDtypeStruct((npk, 128), jnp.float32),
    )(p, g1, dinv, b1t)


def _out_call(p, g2, dinv, w2b, pswap, b2t):
    npk = g2.shape[0]
    return pl.pallas_call(
        _out_body,
        out_shape=jax.ShapeDtypeStruct((npk, 2 * 8), jnp.float32),
    )(p, g2, dinv, w2b, pswap, b2t)


def kernel(x, edge_index, W1, b1, W2, b2):
    n, d = x.shape
    e = edge_index.shape[1]
    h = W1.shape[1]
    c = W2.shape[1]
    n_pad = ((n + 1 + NS * CHUNK - 1) // (NS * CHUNK)) * (NS * CHUNK)
    cpw = (e + NW * CHUNK - 1) // (NW * CHUNK)
    cpw = ((cpw + NBUF - 1) // NBUF) * NBUF
    e_pad = NW * cpw * CHUNK
    npk = n_pad // 8

    ei_pad = jnp.pad(edge_index, ((0, 0), (0, e_pad - e)), constant_values=n)
    src_r = ei_pad[0].reshape(NW, cpw, CHUNK)
    dst_r = ei_pad[1].reshape(NW, cpw, CHUNK)
    x_rs = jnp.pad(x, ((0, n_pad - n), (0, 0))).reshape(npk, 8 * d)

    eye8 = jnp.eye(8, dtype=jnp.float32)
    w1b = jnp.kron(eye8, W1)
    w2b = jnp.kron(eye8, W2)
    pswap = jnp.kron(eye8, jnp.ones((c, c), jnp.float32)
                     - jnp.eye(c, dtype=jnp.float32))
    bcast = jnp.kron(eye8, jnp.ones((1, LANES), jnp.float32))
    b1t = jnp.tile(b1, 8)
    b2t = jnp.tile(b2, 8)

    agg = _make_agg_kernel(n_pad, cpw)
    degp = _make_deg_kernel(n_pad, cpw)(dst_r)
    degp8 = degp.reshape(NC, npk, 8)

    g1, dinv = _g1_call(x_rs, w1b, degp8, bcast)
    p1 = agg(g1.reshape(n_pad, LANES), src_r, dst_r)
    g2 = _g2_call(n // 8, p1.reshape(NC, npk, 128), g1, dinv, b1t)
    p2 = agg(g2.reshape(n_pad, LANES), src_r, dst_r)
    out = _out_call(p2.reshape(NC, npk, 128), g2, dinv, w2b, pswap, b2t)
    return out.reshape(n_pad, c)[:n]


# speedup vs baseline: 52.3087x; 1.0008x over previous
import functools

import jax
import jax.numpy as jnp
from jax import lax
from jax.experimental import pallas as pl
from jax.experimental.pallas import tpu as pltpu
from jax.experimental.pallas import tpu_sc as plsc

NC = 2
NS = 16
NW = NC * NS
LANES = 16
CHUNK = 128
NBUF = 8


def _sc_mesh():
    return plsc.VectorSubcoreMesh(core_axis_name="c", subcore_axis_name="s")


def _make_deg_kernel(n_pad, cpw):
    rps = n_pad // NS

    @functools.partial(
        pl.kernel,
        out_type=jax.ShapeDtypeStruct((NC, n_pad), jnp.float32),
        mesh=_sc_mesh(),
        scratch_types=[
            pltpu.VMEM((cpw, CHUNK), jnp.int32),
            pltpu.VMEM((CHUNK,), jnp.float32),
            pltpu.VMEM_SHARED((n_pad,), jnp.float32),
            pltpu.SemaphoreType.DMA,
        ],
    )
    def deg_kernel(dst_hbm, out_hbm, didx_v, ones_v, deg_sh, sem):
        c = lax.axis_index("c")
        s = lax.axis_index("s")
        w = c * NS + s
        base = pl.multiple_of(s * rps, 8)

        @pl.loop(0, CHUNK, step=LANES)
        def _(i):
            ones_v[pl.ds(i, LANES)] = jnp.zeros((LANES,), jnp.float32)

        for k in range(rps // CHUNK):
            pltpu.sync_copy(ones_v, deg_sh.at[pl.ds(base + k * CHUNK, CHUNK)])

        @pl.loop(0, CHUNK, step=LANES)
        def _(i):
            ones_v[pl.ds(i, LANES)] = jnp.ones((LANES,), jnp.float32)

        pltpu.sync_copy(dst_hbm.at[w], didx_v)
        plsc.subcore_barrier()

        @pl.loop(0, cpw)
        def _(j):
            pltpu.sync_copy(ones_v, deg_sh.at[didx_v.at[j]], add=True)

        plsc.subcore_barrier()
        pltpu.sync_copy(deg_sh.at[pl.ds(base, rps)],
                        out_hbm.at[c, pl.ds(base, rps)])

    return deg_kernel


def _make_agg_kernel(n_pad, cpw):
    rps = n_pad // NS
    assert cpw % NBUF == 0 and cpw // NBUF >= 2
    ngroups = cpw // NBUF

    @functools.partial(
        pl.kernel,
        out_type=jax.ShapeDtypeStruct((NC, n_pad, LANES), jnp.float32),
        mesh=_sc_mesh(),
        scratch_types=[
            pltpu.VMEM((cpw, CHUNK), jnp.int32),
            pltpu.VMEM((cpw, CHUNK), jnp.int32),
            pltpu.VMEM((NBUF, CHUNK, LANES), jnp.float32),
            pltpu.VMEM_SHARED((n_pad, LANES), jnp.float32),
        ] + [pltpu.SemaphoreType.DMA] * (NBUF + 2),
        compiler_params=pltpu.CompilerParams(use_tc_tiling_on_sc=False),
    )
    def agg_kernel(g_hbm, src_hbm, dst_hbm, out_hbm,
                   sidx_v, didx_v, rows_v, agg_sh, *sems):
        c = lax.axis_index("c")
        s = lax.axis_index("s")
        w = c * NS + s
        base = pl.multiple_of(s * rps, 8)

        ci = pltpu.async_copy(src_hbm.at[w], sidx_v, sems[NBUF])
        cd = pltpu.async_copy(dst_hbm.at[w], didx_v, sems[NBUF + 1])

        @pl.loop(0, CHUNK)
        def _(i):
            rows_v[0, i, :] = jnp.zeros((LANES,), jnp.float32)

        for k in range(rps // CHUNK):
            pltpu.sync_copy(rows_v.at[0], agg_sh.at[pl.ds(base + k * CHUNK, CHUNK)])

        ci.wait()
        cd.wait()
        plsc.subcore_barrier()

        for b in range(NBUF):
            pltpu.async_copy(g_hbm.at[sidx_v.at[b]], rows_v.at[b], sems[b])

        @pl.loop(0, ngroups - 1)
        def _(g):
            j = g * NBUF
            for b in range(NBUF):
                pltpu.make_async_copy(
                    g_hbm.at[sidx_v.at[b]], rows_v.at[b], sems[b]).wait()
                pltpu.sync_copy(rows_v.at[b], agg_sh.at[didx_v.at[j + b]],
                                add=True)
                pltpu.async_copy(
                    g_hbm.at[sidx_v.at[j + NBUF + b]], rows_v.at[b], sems[b])

        jl = (ngroups - 1) * NBUF
        for b in range(NBUF):
            pltpu.make_async_copy(
                g_hbm.at[sidx_v.at[b]], rows_v.at[b], sems[b]).wait()
            pltpu.sync_copy(rows_v.at[b], agg_sh.at[didx_v.at[jl + b]],
                            add=True)

        plsc.subcore_barrier()
        pltpu.sync_copy(agg_sh.at[pl.ds(base, rps)],
                        out_hbm.at[c, pl.ds(base, rps)])

    return agg_kernel


def _g1_body(xr_ref, w1b_ref, degp_ref, bcast_ref, g1_ref, dinv_ref):
    deg8 = degp_ref[0] + degp_ref[1] + 1.0
    dinv = jnp.dot(lax.rsqrt(deg8), bcast_ref[...],
                   preferred_element_type=jnp.float32)
    h = jnp.dot(xr_ref[...], w1b_ref[...], preferred_element_type=jnp.float32)
    g1_ref[...] = h * dinv
    dinv_ref[...] = dinv


def _g2_body(zero_row, p_ref, g1_ref, dinv_ref, b1t_ref, g2_ref):
    dinv = dinv_ref[...]
    agg = p_ref[0] + p_ref[1] + g1_ref[...]
    out1 = agg * dinv + b1t_ref[...][None, :]
    g2_ref[...] = jnp.maximum(out1, 0.0) * dinv
    g2_ref[zero_row, 0:LANES] = jnp.zeros((LANES,), jnp.float32)


def _out_body(p_ref, g2_ref, dinv_ref, w2b_ref, pswap_ref, b2t_ref, o_ref):
    agg = (p_ref[0] + p_ref[1] + g2_ref[...]) * dinv_ref[...]
    v = jnp.dot(agg, w2b_ref[...], preferred_element_type=jnp.float32)
    v = v + b2t_ref[...][None, :]
    vs = jnp.dot(v, pswap_ref[...], preferred_element_type=jnp.float32)
    m = jnp.maximum(v, vs)
    lse = m + jnp.log(jnp.exp(v - m) + jnp.exp(vs - m))
    o_ref[...] = v - lse


def _g1_call(x_rs, w1b, degp8, bcast):
    npk = x_rs.shape[0]
    return pl.pallas_call(
        _g1_body,
        out_shape=[
            jax.ShapeDtypeStruct((npk, 128), jnp.float32),
            jax.ShapeDtypeStruct((npk, 128), jnp.float32),
        ],
    )(x_rs, w1b, degp8, bcast)


def _g2_call(zero_row, p, g1, dinv, b1t):
    npk = g1.shape[0]
    return pl.pallas_call(
        functools.partial(_g2_body, zero_row),
        out_shape=jax.ShapeDtypeStruct((npk, 128), jnp.float32),
    )(p, g1, dinv, b1t)


def _out_call(p, g2, dinv, w2b, pswap, b2t):
    npk = g2.shape[0]
    return pl.pallas_call(
        _out_body,
        out_shape=jax.ShapeDtypeStruct((npk, 2 * 8), jnp.float32),
    )(p, g2, dinv, w2b, pswap, b2t)


def kernel(x, edge_index, W1, b1, W2, b2):
    n, d = x.shape
    e = edge_index.shape[1]
    h = W1.shape[1]
    c = W2.shape[1]
    n_pad = ((n + 1 + NS * CHUNK - 1) // (NS * CHUNK)) * (NS * CHUNK)
    cpw = (e + NW * CHUNK - 1) // (NW * CHUNK)
    cpw = ((cpw + NBUF - 1) // NBUF) * NBUF
    e_pad = NW * cpw * CHUNK
    npk = n_pad // 8

    ei_pad = jnp.pad(edge_index, ((0, 0), (0, e_pad - e)), constant_values=n)
    src_r = ei_pad[0].reshape(NW, cpw, CHUNK)
    dst_r = ei_pad[1].reshape(NW, cpw, CHUNK)
    x_rs = jnp.pad(x, ((0, n_pad - n), (0, 0))).reshape(npk, 8 * d)

    eye8 = jnp.eye(8, dtype=jnp.float32)
    w1b = jnp.kron(eye8, W1)
    w2b = jnp.kron(eye8, W2)
    pswap = jnp.kron(eye8, jnp.ones((c, c), jnp.float32)
                     - jnp.eye(c, dtype=jnp.float32))
    bcast = jnp.kron(eye8, jnp.ones((1, LANES), jnp.float32))
    b1t = jnp.tile(b1, 8)
    b2t = jnp.tile(b2, 8)

    agg = _make_agg_kernel(n_pad, cpw)
    degp = _make_deg_kernel(n_pad, cpw)(dst_r)
    degp8 = degp.reshape(NC, npk, 8)

    g1, dinv = _g1_call(x_rs, w1b, degp8, bcast)
    p1 = agg(g1.reshape(n_pad, LANES), src_r, dst_r)
    g2 = _g2_call(n // 8, p1.reshape(NC, npk, 128), g1, dinv, b1t)
    p2 = agg(g2.reshape(n_pad, LANES), src_r, dst_r)
    out = _out_call(p2.reshape(NC, npk, 128), g2, dinv, w2b, pswap, b2t)
    return out.reshape(n_pad, c)[:n]
